# Initial kernel scaffold; baseline (speedup 1.0000x reference)
import jax
import jax.numpy as jnp
from jax.experimental import pallas as pl


def kernel(x, edge_index):
    raise NotImplementedError("write your pallas kernel here")


# speedup vs baseline: 1.0349x; 1.0349x over previous
import jax
import jax.numpy as jnp
from jax.experimental import pallas as pl


def _merge_body(x_ref, summed_ref, cnt_ref, out_ref):
    x = x_ref[...]
    s = summed_ref[...]
    c = cnt_ref[...]
    out = s / jnp.maximum(c, 1.0)
    out_ref[...] = jnp.where(x == 0, out, x)


def _merge(x, summed, cnt, interpret=False):
    T, N, F = x.shape
    BN = 1000
    grid = (N // BN,)
    cnt3 = jnp.broadcast_to(cnt.reshape(1, N, 1), (1, N, F))
    return pl.pallas_call(
        _merge_body,
        grid=grid,
        in_specs=[
            pl.BlockSpec((T, BN, F), lambda i: (0, i, 0)),
            pl.BlockSpec((T, BN, F), lambda i: (0, i, 0)),
            pl.BlockSpec((1, BN, F), lambda i: (0, i, 0)),
        ],
        out_specs=pl.BlockSpec((T, BN, F), lambda i: (0, i, 0)),
        out_shape=jax.ShapeDtypeStruct((T, N, F), x.dtype),
        interpret=interpret,
    )(x, summed, cnt3)


def kernel(x, edge_index):
    n_nodes = x.shape[-2]
    src = edge_index[0].astype(jnp.int64)
    dst = edge_index[1].astype(jnp.int64)
    code = jnp.sort(src * n_nodes + dst)
    valid = jnp.concatenate([jnp.ones((1,), dtype=bool), code[1:] != code[:-1]])
    src = (code // n_nodes).astype(jnp.int32)
    dst = (code % n_nodes).astype(jnp.int32)

    def cond(carry):
        xc, i = carry
        return jnp.any(xc == 0) & (i < 20)

    def body(carry):
        xc, i = carry
        node_nonzero = jnp.sum(xc != 0, axis=(0, 2)) != 0
        em = (node_nonzero[src] & valid).astype(xc.dtype)
        msgs = xc[:, src, :] * em[None, :, None]
        summed = jnp.zeros_like(xc).at[:, dst, :].add(msgs)
        cnt = jnp.zeros((n_nodes,), xc.dtype).at[dst].add(em)
        return _merge(xc, summed, cnt), i + 1

    x, _ = jax.lax.while_loop(cond, body, (x, jnp.int32(0)))
    return x


# speedup vs baseline: 2.5806x; 2.4936x over previous
import functools

import jax
import jax.numpy as jnp
from jax import lax
from jax.experimental import pallas as pl
from jax.experimental.pallas import tpu as pltpu
from jax.experimental.pallas import tpu_sc as plsc

N_NODES = 10000
N_EDGES = 160000
NW = 32
RPT = 320
NPAD = NW * RPT
DUMMY = N_NODES
SRC_PAD = N_NODES + 8
CHUNK = 256
FH = 256


def _scalar(ref):
    return jnp.max(ref[...])


SLAB = RPT + 8


def _sc_body(xna, xnb, srcs, dsts, zrow, loc, hic, dlo_a, outa, outb,
             sbuf, dbuf, rowbuf, acc, b16a, b16b, b16c, gsem):
    sid = lax.axis_index("s")
    wid = sid * 2 + lax.axis_index("c")
    slab0 = sid * SLAB

    pltpu.sync_copy(loc.at[pl.ds(wid * 16, 16)], b16a)
    pltpu.sync_copy(hic.at[pl.ds(wid * 16, 16)], b16b)
    pltpu.sync_copy(dlo_a.at[pl.ds(wid * 16, 16)], b16c)
    c_lo = _scalar(b16a)
    c_hi = _scalar(b16b)
    dlo = pl.multiple_of(_scalar(b16c), RPT)

    for h in range(2):
        xnh = xna if h == 0 else xnb
        outh = outa if h == 0 else outb
        pltpu.sync_copy(zrow, acc)

        def group_body(j, carry):
            sv = sbuf[pl.ds(16 * j, 16)]
            dv = dbuf[pl.ds(16 * j, 16)]
            m = (dv >= dlo) & (dv < dlo + RPT)
            didx = jnp.where(m, dv - dlo, RPT)
            pltpu.async_copy(xnh.at[sv], rowbuf, gsem).wait()
            for k in range(16):
                onek = lax.iota(jnp.int32, 16) == k
                dk = jnp.max(jnp.where(onek, didx, 0))
                rbase = pl.multiple_of(dk * FH, 16)
                for f in range(FH // 16):
                    plsc.addupdate(acc.at[pl.ds(rbase + 16 * f, 16)],
                                   rowbuf[k, pl.ds(16 * f, 16)])
            return carry

        def chunk_body(c, carry):
            base = c * CHUNK
            pltpu.sync_copy(srcs.at[pl.ds(base, CHUNK)], sbuf)
            pltpu.sync_copy(dsts.at[pl.ds(base, CHUNK)], dbuf)
            return lax.fori_loop(0, CHUNK // 16, group_body, carry)

        lax.fori_loop(c_lo, c_hi, chunk_body, 0)
        pltpu.sync_copy(acc.at[pl.ds(0, RPT * FH)],
                        outh.at[pl.ds(dlo * FH, RPT * FH)])


@functools.partial(jax.jit, static_argnums=())
def _sc_summed(xna, xnb, srcs, dsts, zrow, loc, hic, dlo_a):
    mesh = plsc.VectorSubcoreMesh(core_axis_name="c", subcore_axis_name="s")
    f = pl.kernel(
        _sc_body,
        out_type=[
            jax.ShapeDtypeStruct((NPAD * FH,), jnp.float32),
            jax.ShapeDtypeStruct((NPAD * FH,), jnp.float32),
        ],
        mesh=mesh,
        compiler_params=pltpu.CompilerParams(needs_layout_passes=False),
        scratch_types=[
            pltpu.VMEM((CHUNK,), jnp.int32),
            pltpu.VMEM((CHUNK,), jnp.int32),
            pltpu.VMEM((16, FH), jnp.float32),
            pltpu.VMEM((SLAB * FH,), jnp.float32),
            pltpu.VMEM((16,), jnp.int32),
            pltpu.VMEM((16,), jnp.int32),
            pltpu.VMEM((16,), jnp.int32),
            pltpu.SemaphoreType.DMA,
        ],
    )
    return f(xna, xnb, srcs, dsts, zrow, loc, hic, dlo_a)


def _merge_body(x_ref, summed_ref, cnt_ref, out_ref):
    x = x_ref[...]
    s = summed_ref[...]
    c = cnt_ref[:, :1]
    out = s / jnp.maximum(c, 1.0)
    out_ref[...] = jnp.where(x == 0, out, x)


def _merge(xt, summed, cnt):
    N, F = xt.shape
    BN = 1000
    cnt2 = jnp.broadcast_to(cnt.reshape(N, 1), (N, 128))
    return pl.pallas_call(
        _merge_body,
        grid=(N // BN,),
        in_specs=[
            pl.BlockSpec((BN, F), lambda i: (i, 0)),
            pl.BlockSpec((BN, F), lambda i: (i, 0)),
            pl.BlockSpec((BN, 128), lambda i: (i, 0)),
        ],
        out_specs=pl.BlockSpec((BN, F), lambda i: (i, 0)),
        out_shape=jax.ShapeDtypeStruct((N, F), xt.dtype),
    )(xt, summed, cnt2)


def kernel(x, edge_index):
    T, N, F = x.shape
    src = edge_index[0].astype(jnp.int64)
    dst = edge_index[1].astype(jnp.int64)
    code = jnp.sort(dst * N + src)
    valid = jnp.concatenate([jnp.ones((1,), dtype=bool), code[1:] != code[:-1]])
    srcs = (code % N).astype(jnp.int32)
    dsts = (code // N).astype(jnp.int32)

    bounds = jnp.arange(NW + 1, dtype=jnp.int32) * RPT
    ptr = jnp.searchsorted(dsts, bounds).astype(jnp.int32)
    loc = jnp.broadcast_to((ptr[:-1] // CHUNK)[:, None], (NW, 16)).reshape(-1)
    hic = jnp.broadcast_to(((ptr[1:] + CHUNK - 1) // CHUNK)[:, None],
                           (NW, 16)).reshape(-1)
    dlo_a = jnp.broadcast_to((bounds[:-1])[:, None], (NW, 16)).reshape(-1)
    zrow = jnp.zeros((SLAB * FH,), jnp.float32)

    xt0 = x.transpose(1, 0, 2).reshape(N, T * F)

    def cond(carry):
        xt, i = carry
        return jnp.any(xt == 0) & (i < 20)

    def body(carry):
        xt, i = carry
        node_nonzero = jnp.any(xt != 0, axis=1)
        em = node_nonzero[srcs] & valid
        src_eff = jnp.where(em, srcs, DUMMY)
        cnt = jnp.zeros((N,), jnp.float32).at[dsts].add(
            em.astype(jnp.float32), mode="drop")
        xp = jnp.pad(xt, ((0, SRC_PAD - N), (0, 0)))
        sa, sb = _sc_summed(xp[:, :FH], xp[:, FH:], src_eff, dsts,
                            zrow, loc, hic, dlo_a)
        summed = jnp.concatenate([sa.reshape(NPAD, FH)[:N],
                                  sb.reshape(NPAD, FH)[:N]], axis=1)
        return _merge(xt, summed, cnt), i + 1

    xt, _ = lax.while_loop(cond, body, (xt0, jnp.int32(0)))
    return xt.reshape(N, T, F).transpose(1, 0, 2)
